# baseline (speedup 1.0000x reference)
import dataclasses
import functools

import jax
import jax.numpy as jnp
from jax import lax
from jax.experimental import pallas as pl
from jax.experimental.pallas import tpu as pltpu
from jax.experimental.pallas import tpu_sc as plsc

N_NODES = 10000
NP = 10240
R_TILE = 320
N_WORKERS = 32
HSTEP = 3.0
BR = 1024


@functools.lru_cache(maxsize=None)
def _seg_sums_sc(F, K):
    mesh = plsc.VectorSubcoreMesh(core_axis_name="c", subcore_axis_name="s")
    cp = pltpu.CompilerParams(needs_layout_passes=False,
                              use_tc_tiling_on_sc=False)

    @functools.partial(
        pl.kernel,
        out_type=jax.ShapeDtypeStruct((NP, 2 * F), jnp.float32),
        mesh=mesh,
        compiler_params=cp,
        scratch_types=[
            pltpu.VMEM((K,), jnp.int32),
            pltpu.VMEM((K,), jnp.float32),
            pltpu.VMEM((K,), jnp.int32),
            pltpu.VMEM((K, F), jnp.float32),
            pltpu.VMEM((R_TILE, 2 * F), jnp.float32),
            pltpu.VMEM((48,), jnp.int32),
            pltpu.SemaphoreType.DMA,
        ],
    )
    def sc_kernel(z_hbm, src_hbm, u_hbm, dst_hbm, off_hbm, g_hbm,
                  srcv, uv, dv, zbuf, acc, offv, sem):
        wid = lax.axis_index("s") * 2 + lax.axis_index("c")
        r0 = wid * R_TILE
        pltpu.sync_copy(off_hbm, offv)
        iota = lax.iota(jnp.int32, 16)
        lo = jnp.int32(0)
        hi = jnp.int32(0)
        for g in range(3):
            vals = offv[pl.ds(g * 16, 16)]
            idx = iota + (g * 16)
            lo = lo + jnp.sum(jnp.where(idx == wid, vals, 0))
            hi = hi + jnp.sum(jnp.where(idx == wid + 1, vals, 0))

        zeros16 = jnp.zeros((16,), jnp.float32)

        @pl.loop(0, R_TILE)
        def _(r):
            for f in range(0, 2 * F, 16):
                acc[r, pl.ds(f, 16)] = zeros16

        c0 = lo // K
        c1 = (hi + (K - 1)) // K

        def chunk_body(c, carry):
            base = c * K
            pltpu.sync_copy(src_hbm.at[pl.ds(base, K)], srcv)
            pltpu.sync_copy(u_hbm.at[pl.ds(base, K)], uv)
            pltpu.sync_copy(dst_hbm.at[pl.ds(base, K)], dv)
            pltpu.async_copy(z_hbm.at[srcv], zbuf, sem).wait()

            @pl.loop(0, K, step=16)
            def _(e):
                eg = (base + e) + iota
                valid = (eg >= lo) & (eg < hi)
                u16 = uv[pl.ds(e, 16)]
                d16 = dv[pl.ds(e, 16)] - r0
                w0 = 1.0 - u16
                e16 = iota + e
                for f in range(F):
                    fidx = jnp.full((16,), f, jnp.int32)
                    fidx2 = jnp.full((16,), F + f, jnp.int32)
                    zc = plsc.load_gather(zbuf, [e16, fidx])
                    plsc.addupdate_scatter(acc, [d16, fidx], w0 * zc,
                                           mask=valid)
                    plsc.addupdate_scatter(acc, [d16, fidx2], u16 * zc,
                                           mask=valid)
            return carry

        lax.fori_loop(c0, c1, chunk_body, jnp.int32(0))
        pltpu.sync_copy(acc, g_hbm.at[pl.ds(r0, R_TILE)])

    return sc_kernel


def _mm(a, b):
    return jnp.dot(a, b, preferred_element_type=jnp.float32)


def _conv_base(z_ref, g_ref, wc_ref, rt_ref, b_ref, inv_ref):
    return (inv_ref[...] * _mm(g_ref[...], wc_ref[...])
            + _mm(z_ref[...], rt_ref[...]) + b_ref[...])


def _row_spec(width):
    return pl.BlockSpec((BR, width), lambda i: (i, 0))


def _full_spec(shape):
    return pl.BlockSpec(shape, lambda i: (0, 0))


def _tc_call(body, n_out, fin, gw, hout, extra_row_ins=0):
    in_specs = [
        _row_spec(fin),
        _row_spec(gw),
        _full_spec((gw, hout)),
        _full_spec((fin, hout)),
        _full_spec((1, hout)),
        pl.BlockSpec((BR, 1), lambda i: (i, 0)),
    ] + [_row_spec(hout)] * extra_row_ins
    out_shape = [jax.ShapeDtypeStruct((NP, hout), jnp.float32)] * n_out
    out_specs = [_row_spec(hout)] * n_out
    if n_out == 1:
        out_shape, out_specs = out_shape[0], out_specs[0]
    return pl.pallas_call(
        body,
        grid=(NP // BR,),
        in_specs=in_specs,
        out_specs=out_specs,
        out_shape=out_shape,
    )


@functools.lru_cache(maxsize=None)
def _tc_conv_tanh(fin, gw, hout):
    def body(z, g, wc, rt, b, inv, o):
        o[...] = jnp.tanh(_conv_base(z, g, wc, rt, b, inv))
    return _tc_call(body, 1, fin, gw, hout)


@functools.lru_cache(maxsize=None)
def _tc_conv_plain(fin, gw, hout):
    def body(z, g, wc, rt, b, inv, o):
        o[...] = _conv_base(z, g, wc, rt, b, inv)
    return _tc_call(body, 1, fin, gw, hout)


@functools.lru_cache(maxsize=None)
def _tc_conv_rk4(fin, gw, hout, ci, wi):
    def body(z, g, wc, rt, b, inv, h, accp, oz, oa):
        k = _conv_base(z, g, wc, rt, b, inv)
        oz[...] = h[...] + ci * k
        oa[...] = accp[...] + wi * k
    return _tc_call(body, 2, fin, gw, hout, extra_row_ins=2)


@functools.lru_cache(maxsize=None)
def _tc_conv2_logsoftmax(fin, gw, hout, ncls):
    def body(z, g, wc, rt, b, inv, o):
        t = jnp.tanh(_conv_base(z, g, wc, rt, b, inv))
        col = lax.broadcasted_iota(jnp.int32, t.shape, 1)
        m = col < ncls
        tm = jnp.where(m, t, jnp.float32(-1e30))
        mx = jnp.max(tm, axis=1, keepdims=True)
        ex = jnp.where(m, jnp.exp(t - mx), 0.0)
        lse = jnp.log(jnp.sum(ex, axis=1, keepdims=True))
        o[...] = t - mx - lse
    return _tc_call(body, 1, fin, gw, hout)


def _tc_invdeg():
    def body(a_ref, b_ref, o_ref):
        cnt = (b_ref[...] - a_ref[...]).astype(jnp.float32)
        o_ref[...] = 1.0 / jnp.maximum(cnt, 1.0)
    return pl.pallas_call(
        body,
        grid=(NP // BR,),
        in_specs=[pl.BlockSpec((BR, 1), lambda i: (i, 0))] * 2,
        out_specs=pl.BlockSpec((BR, 1), lambda i: (i, 0)),
        out_shape=jax.ShapeDtypeStruct((NP, 1), jnp.float32),
    )


def kernel(x, edge_index, edge_attr, W1, root1, b1, Wa, roota, ba,
           Wb, rootb, bb, W2, root2, b2):
    E = edge_index.shape[1]
    src = edge_index[0].astype(jnp.int32)
    dst = edge_index[1].astype(jnp.int32)
    u = edge_attr[:, 0]

    perm = jnp.argsort(dst)
    dst_s = dst[perm]
    src_s = src[perm]
    u_s = u[perm]

    KPAD = 1024
    Ep = ((E + KPAD - 1) // KPAD) * KPAD
    pad = Ep - E
    src_p = jnp.pad(src_s, (0, pad))
    dst_p = jnp.pad(dst_s, (0, pad))
    u_p = jnp.pad(u_s, (0, pad))

    seg = jnp.searchsorted(
        dst_s, jnp.arange(N_NODES + 1, dtype=jnp.int32), side="left"
    ).astype(jnp.int32)
    rb = jnp.minimum(jnp.arange(33, dtype=jnp.int32) * R_TILE, N_NODES)
    off48 = jnp.pad(seg[rb], (0, 15))

    seg_a = jnp.pad(seg[:N_NODES], (0, NP - N_NODES)).reshape(NP, 1)
    seg_b = jnp.pad(seg[1:], (0, NP - N_NODES)).reshape(NP, 1)
    invdeg = _tc_invdeg()(seg_a, seg_b)

    xp = jnp.pad(x, ((0, NP - N_NODES), (0, 0)))

    D, H = x.shape[1], W1.shape[2]
    NCLS = W2.shape[2]
    HP = 128
    wc1 = jnp.concatenate([W1[0], W1[1]], axis=0)
    wca = jnp.concatenate([Wa[0], Wa[1]], axis=0)
    wcb = jnp.concatenate([Wb[0], Wb[1]], axis=0)
    wc2 = jnp.pad(jnp.concatenate([W2[0], W2[1]], axis=0),
                  ((0, 0), (0, HP - NCLS)))
    root2p = jnp.pad(root2, ((0, 0), (0, HP - NCLS)))
    b2p = jnp.pad(b2, (0, HP - NCLS)).reshape(1, HP)
    b1r = b1.reshape(1, H)
    bar = ba.reshape(1, H)
    bbr = bb.reshape(1, H)

    sc_d = _seg_sums_sc(D, 256)
    sc_h = _seg_sums_sc(H, 1024)
    edges = (src_p, u_p, dst_p, off48)

    g = sc_d(xp, *edges)
    h = _tc_conv_tanh(D, 2 * D, H)(xp, g, wc1, root1, b1r, invdeg)

    def conv_a(z):
        gz = sc_h(z, *edges)
        return _tc_conv_plain(H, 2 * H, H)(z, gz, wca, roota, bar, invdeg)

    def conv_b_rk4(z, accp, ci, wi):
        gz = sc_h(z, *edges)
        return _tc_conv_rk4(H, 2 * H, H, ci, wi)(
            z, gz, wcb, rootb, bbr, invdeg, h, accp)

    z2, acc1 = conv_b_rk4(conv_a(h), h, 0.5 * HSTEP, HSTEP / 6.0)
    z3, acc2 = conv_b_rk4(conv_a(z2), acc1, 0.5 * HSTEP, 2.0 * HSTEP / 6.0)
    z4, acc3 = conv_b_rk4(conv_a(z3), acc2, HSTEP, 2.0 * HSTEP / 6.0)
    _, hfin = conv_b_rk4(conv_a(z4), acc3, 0.0, HSTEP / 6.0)

    g2 = sc_h(hfin, *edges)
    out = _tc_conv2_logsoftmax(H, 2 * H, HP, NCLS)(
        hfin, g2, wc2, root2p, b2p, invdeg)
    return out[:N_NODES, :NCLS]


# speedup vs baseline: 3.8549x; 3.8549x over previous
import dataclasses
import functools

import jax
import jax.numpy as jnp
from jax import lax
from jax.experimental import pallas as pl
from jax.experimental.pallas import tpu as pltpu
from jax.experimental.pallas import tpu_sc as plsc

N_NODES = 10000
NP = 10240
R_TILE = 320
N_WORKERS = 32
HSTEP = 3.0
BR = 1024


@functools.lru_cache(maxsize=None)
def _seg_sums_sc(F, K):
    assert K & (K - 1) == 0
    LOGK = K.bit_length() - 1
    NG = F // 16
    mesh = plsc.VectorSubcoreMesh(core_axis_name="c", subcore_axis_name="s")
    cp = pltpu.CompilerParams(needs_layout_passes=False,
                              use_tc_tiling_on_sc=False)

    @functools.partial(
        pl.kernel,
        out_type=jax.ShapeDtypeStruct((NP, 2 * F), jnp.float32),
        mesh=mesh,
        compiler_params=cp,
        scratch_types=[
            pltpu.VMEM((K,), jnp.int32),
            pltpu.VMEM((K, F), jnp.float32),
            pltpu.VMEM((R_TILE, 2 * F), jnp.float32),
            pltpu.VMEM((K + 16,), jnp.float32),
            pltpu.VMEM((R_TILE + 24,), jnp.int32),
            pltpu.SemaphoreType.DMA,
        ],
    )
    def sc_kernel(z_hbm, src_hbm, u_hbm, seg_hbm, g_hbm,
                  srcv, zbuf, acc, useg, segs, sem):
        wid = lax.axis_index("s") * 2 + lax.axis_index("c")
        r0 = wid * R_TILE
        pltpu.sync_copy(seg_hbm.at[pl.ds(r0, R_TILE + 8)],
                        segs.at[pl.ds(0, R_TILE + 8)])

        zeros16 = jnp.zeros((16,), jnp.float32)

        ones16 = jnp.ones((16,), jnp.float32)

        def edge_body(base):
            def body(j, accs):
                el = j - base
                uvec = plsc.load_gather(useg, [jnp.broadcast_to(el, (16,))])
                w0 = ones16 - uvec
                new = []
                for g in range(NG):
                    zv = zbuf[el, pl.ds(g * 16, 16)]
                    new.append(accs[g] + w0 * zv)
                for g in range(NG):
                    zv = zbuf[el, pl.ds(g * 16, 16)]
                    new.append(accs[NG + g] + uvec * zv)
                return tuple(new)
            return body

        def row_body(r, carry):
            loaded, s_cur = carry
            e2 = segs[pl.ds(r + 1, 16)][0]

            def chunk_step(state):
                j, loaded_in = state[0], state[1]
                accs = state[2:]
                cj = lax.shift_right_logical(j, LOGK)
                base = cj * K

                @pl.when(cj != loaded_in)
                def _():
                    pltpu.sync_copy(src_hbm.at[pl.ds(base, K)], srcv)
                    pltpu.sync_copy(u_hbm.at[pl.ds(base, K)],
                                    useg.at[pl.ds(0, K)])
                    pltpu.async_copy(z_hbm.at[srcv], zbuf, sem).wait()

                jend = jnp.minimum(e2, base + K)
                accs = lax.fori_loop(j, jend, edge_body(base), accs)
                return (jend, cj) + accs

            init = (s_cur, loaded) + (zeros16,) * (2 * NG)
            out = lax.while_loop(lambda st: st[0] < e2, chunk_step, init)
            for g in range(2 * NG):
                acc[r, pl.ds(g * 16, 16)] = out[2 + g]
            return out[1], e2

        lax.fori_loop(0, R_TILE, row_body,
                      (jnp.int32(-1), segs[pl.ds(0, 16)][0]))
        pltpu.sync_copy(acc, g_hbm.at[pl.ds(r0, R_TILE)])

    return sc_kernel


def _mm(a, b):
    return jnp.dot(a, b, preferred_element_type=jnp.float32)


def _conv_base(z_ref, g_ref, wc_ref, rt_ref, b_ref, inv_ref):
    return (inv_ref[...] * _mm(g_ref[...], wc_ref[...])
            + _mm(z_ref[...], rt_ref[...]) + b_ref[...])


def _row_spec(width):
    return pl.BlockSpec((BR, width), lambda i: (i, 0))


def _full_spec(shape):
    return pl.BlockSpec(shape, lambda i: (0, 0))


def _tc_call(body, n_out, fin, gw, hout, extra_row_ins=0):
    in_specs = [
        _row_spec(fin),
        _row_spec(gw),
        _full_spec((gw, hout)),
        _full_spec((fin, hout)),
        _full_spec((1, hout)),
        pl.BlockSpec((BR, 1), lambda i: (i, 0)),
    ] + [_row_spec(hout)] * extra_row_ins
    out_shape = [jax.ShapeDtypeStruct((NP, hout), jnp.float32)] * n_out
    out_specs = [_row_spec(hout)] * n_out
    if n_out == 1:
        out_shape, out_specs = out_shape[0], out_specs[0]
    return pl.pallas_call(
        body,
        grid=(NP // BR,),
        in_specs=in_specs,
        out_specs=out_specs,
        out_shape=out_shape,
    )


@functools.lru_cache(maxsize=None)
def _tc_conv_tanh(fin, gw, hout):
    def body(z, g, wc, rt, b, inv, o):
        o[...] = jnp.tanh(_conv_base(z, g, wc, rt, b, inv))
    return _tc_call(body, 1, fin, gw, hout)


@functools.lru_cache(maxsize=None)
def _tc_conv_plain(fin, gw, hout):
    def body(z, g, wc, rt, b, inv, o):
        o[...] = _conv_base(z, g, wc, rt, b, inv)
    return _tc_call(body, 1, fin, gw, hout)


@functools.lru_cache(maxsize=None)
def _tc_conv_rk4(fin, gw, hout, ci, wi):
    def body(z, g, wc, rt, b, inv, h, accp, oz, oa):
        k = _conv_base(z, g, wc, rt, b, inv)
        oz[...] = h[...] + ci * k
        oa[...] = accp[...] + wi * k
    return _tc_call(body, 2, fin, gw, hout, extra_row_ins=2)


@functools.lru_cache(maxsize=None)
def _tc_conv2_logsoftmax(fin, gw, hout, ncls):
    def body(z, g, wc, rt, b, inv, o):
        t = jnp.tanh(_conv_base(z, g, wc, rt, b, inv))
        col = lax.broadcasted_iota(jnp.int32, t.shape, 1)
        m = col < ncls
        tm = jnp.where(m, t, jnp.float32(-1e30))
        mx = jnp.max(tm, axis=1, keepdims=True)
        ex = jnp.where(m, jnp.exp(t - mx), 0.0)
        lse = jnp.log(jnp.sum(ex, axis=1, keepdims=True))
        o[...] = t - mx - lse
    return _tc_call(body, 1, fin, gw, hout)


def _tc_invdeg():
    def body(a_ref, b_ref, o_ref):
        cnt = (b_ref[...] - a_ref[...]).astype(jnp.float32)
        o_ref[...] = 1.0 / jnp.maximum(cnt, 1.0)
    return pl.pallas_call(
        body,
        grid=(NP // BR,),
        in_specs=[pl.BlockSpec((BR, 1), lambda i: (i, 0))] * 2,
        out_specs=pl.BlockSpec((BR, 1), lambda i: (i, 0)),
        out_shape=jax.ShapeDtypeStruct((NP, 1), jnp.float32),
    )


def kernel(x, edge_index, edge_attr, W1, root1, b1, Wa, roota, ba,
           Wb, rootb, bb, W2, root2, b2):
    E = edge_index.shape[1]
    src = edge_index[0].astype(jnp.int32)
    dst = edge_index[1].astype(jnp.int32)
    u = edge_attr[:, 0]

    perm = jnp.argsort(dst)
    dst_s = dst[perm]
    src_s = src[perm]
    u_s = u[perm]

    KPAD = 1024
    Ep = ((E + KPAD - 1) // KPAD) * KPAD
    pad = Ep - E
    src_p = jnp.pad(src_s, (0, pad))
    u_p = jnp.pad(u_s, (0, pad))

    seg = jnp.searchsorted(
        dst_s, jnp.arange(N_NODES + 1, dtype=jnp.int32), side="left"
    ).astype(jnp.int32)
    seg_p = jnp.pad(seg, (0, NP + 8 - (N_NODES + 1)),
                    constant_values=E)

    seg_a = jnp.pad(seg[:N_NODES], (0, NP - N_NODES)).reshape(NP, 1)
    seg_b = jnp.pad(seg[1:], (0, NP - N_NODES)).reshape(NP, 1)
    invdeg = _tc_invdeg()(seg_a, seg_b)

    xp = jnp.pad(x, ((0, NP - N_NODES), (0, 0)))

    D, H = x.shape[1], W1.shape[2]
    NCLS = W2.shape[2]
    HP = 128
    wc1 = jnp.concatenate([W1[0], W1[1]], axis=0)
    wca = jnp.concatenate([Wa[0], Wa[1]], axis=0)
    wcb = jnp.concatenate([Wb[0], Wb[1]], axis=0)
    wc2 = jnp.pad(jnp.concatenate([W2[0], W2[1]], axis=0),
                  ((0, 0), (0, HP - NCLS)))
    root2p = jnp.pad(root2, ((0, 0), (0, HP - NCLS)))
    b2p = jnp.pad(b2, (0, HP - NCLS)).reshape(1, HP)
    b1r = b1.reshape(1, H)
    bar = ba.reshape(1, H)
    bbr = bb.reshape(1, H)

    sc_d = _seg_sums_sc(D, 256)
    sc_h = _seg_sums_sc(H, 1024)
    edges = (src_p, u_p, seg_p)

    g = sc_d(xp, *edges)
    h = _tc_conv_tanh(D, 2 * D, H)(xp, g, wc1, root1, b1r, invdeg)

    def conv_a(z):
        gz = sc_h(z, *edges)
        return _tc_conv_plain(H, 2 * H, H)(z, gz, wca, roota, bar, invdeg)

    def conv_b_rk4(z, accp, ci, wi):
        gz = sc_h(z, *edges)
        return _tc_conv_rk4(H, 2 * H, H, ci, wi)(
            z, gz, wcb, rootb, bbr, invdeg, h, accp)

    z2, acc1 = conv_b_rk4(conv_a(h), h, 0.5 * HSTEP, HSTEP / 6.0)
    z3, acc2 = conv_b_rk4(conv_a(z2), acc1, 0.5 * HSTEP, 2.0 * HSTEP / 6.0)
    z4, acc3 = conv_b_rk4(conv_a(z3), acc2, HSTEP, 2.0 * HSTEP / 6.0)
    _, hfin = conv_b_rk4(conv_a(z4), acc3, 0.0, HSTEP / 6.0)

    g2 = sc_h(hfin, *edges)
    out = _tc_conv2_logsoftmax(H, 2 * H, HP, NCLS)(
        hfin, g2, wc2, root2p, b2p, invdeg)
    return out[:N_NODES, :NCLS]


# speedup vs baseline: 14.1544x; 3.6718x over previous
import dataclasses
import functools

import jax
import jax.numpy as jnp
from jax import lax
from jax.experimental import pallas as pl
from jax.experimental.pallas import tpu as pltpu
from jax.experimental.pallas import tpu_sc as plsc

N_NODES = 10000
NP = 10240
R_TILE = 320
N_WORKERS = 32
HSTEP = 3.0
BR = 1024


@functools.lru_cache(maxsize=None)
def _seg_sums_sc(F, K):
    assert K & (K - 1) == 0
    LOGK = K.bit_length() - 1
    NG = F // 16
    mesh = plsc.VectorSubcoreMesh(core_axis_name="c", subcore_axis_name="s")
    cp = pltpu.CompilerParams(needs_layout_passes=False,
                              use_tc_tiling_on_sc=False)

    @functools.partial(
        pl.kernel,
        out_type=jax.ShapeDtypeStruct((NP, 2 * F), jnp.float32),
        mesh=mesh,
        compiler_params=cp,
        scratch_types=[
            pltpu.VMEM((K,), jnp.int32),
            pltpu.VMEM((K, F), jnp.float32),
            pltpu.VMEM((R_TILE, 2 * F), jnp.float32),
            pltpu.VMEM((K + 16,), jnp.float32),
            pltpu.VMEM((R_TILE + 24,), jnp.int32),
            pltpu.SemaphoreType.DMA,
        ],
    )
    def sc_kernel(z_hbm, src_hbm, u_hbm, seg_hbm, g_hbm,
                  srcv, zbuf, acc, useg, segs, sem):
        wid = lax.axis_index("s") * 2 + lax.axis_index("c")
        r0 = wid * R_TILE
        pltpu.sync_copy(seg_hbm.at[pl.ds(r0, R_TILE + 8)],
                        segs.at[pl.ds(0, R_TILE + 8)])

        zeros16 = jnp.zeros((16,), jnp.float32)

        ones16 = jnp.ones((16,), jnp.float32)

        def edge_body(base):
            def body(j, accs):
                el = j - base
                uvec = plsc.load_gather(useg, [jnp.broadcast_to(el, (16,))])
                w0 = ones16 - uvec
                new = []
                for g in range(NG):
                    zv = zbuf[el, pl.ds(g * 16, 16)]
                    new.append(accs[g] + w0 * zv)
                for g in range(NG):
                    zv = zbuf[el, pl.ds(g * 16, 16)]
                    new.append(accs[NG + g] + uvec * zv)
                return tuple(new)
            return body

        def row_body(r, carry):
            loaded, s_cur = carry
            e2 = segs[pl.ds(r + 1, 16)][0]

            def chunk_step(state):
                j, loaded_in = state[0], state[1]
                accs = state[2:]
                cj = lax.shift_right_logical(j, LOGK)
                base = cj * K

                @pl.when(cj != loaded_in)
                def _():
                    pltpu.sync_copy(src_hbm.at[pl.ds(base, K)], srcv)
                    pltpu.sync_copy(u_hbm.at[pl.ds(base, K)],
                                    useg.at[pl.ds(0, K)])
                    pltpu.async_copy(z_hbm.at[srcv], zbuf, sem).wait()

                jend = jnp.minimum(e2, base + K)
                accs = lax.fori_loop(j, jend, edge_body(base), accs)
                return (jend, cj) + accs

            init = (s_cur, loaded) + (zeros16,) * (2 * NG)
            out = lax.while_loop(lambda st: st[0] < e2, chunk_step, init)
            for g in range(2 * NG):
                acc[r, pl.ds(g * 16, 16)] = out[2 + g]
            return out[1], e2

        lax.fori_loop(0, R_TILE, row_body,
                      (jnp.int32(-1), segs[pl.ds(0, 16)][0]))
        pltpu.sync_copy(acc, g_hbm.at[pl.ds(r0, R_TILE)])

    return sc_kernel


def _mm(a, b):
    ah = a.astype(jnp.bfloat16)
    al = (a - ah.astype(jnp.float32)).astype(jnp.bfloat16)
    bh = b.astype(jnp.bfloat16)
    bl = (b - bh.astype(jnp.float32)).astype(jnp.bfloat16)
    d = functools.partial(jnp.dot, preferred_element_type=jnp.float32)
    return d(ah, bh) + (d(al, bh) + d(ah, bl))


def _conv_base(z_ref, g_ref, wc_ref, rt_ref, b_ref, inv_ref):
    return (inv_ref[...] * _mm(g_ref[...], wc_ref[...])
            + _mm(z_ref[...], rt_ref[...]) + b_ref[...])


def _row_spec(width):
    return pl.BlockSpec((BR, width), lambda i: (i, 0))


def _full_spec(shape):
    return pl.BlockSpec(shape, lambda i: (0, 0))


def _tc_call(body, n_out, fin, gw, hout, extra_row_ins=0):
    in_specs = [
        _row_spec(fin),
        _row_spec(gw),
        _full_spec((gw, hout)),
        _full_spec((fin, hout)),
        _full_spec((1, hout)),
        pl.BlockSpec((BR, 1), lambda i: (i, 0)),
    ] + [_row_spec(hout)] * extra_row_ins
    out_shape = [jax.ShapeDtypeStruct((NP, hout), jnp.float32)] * n_out
    out_specs = [_row_spec(hout)] * n_out
    if n_out == 1:
        out_shape, out_specs = out_shape[0], out_specs[0]
    return pl.pallas_call(
        body,
        grid=(NP // BR,),
        in_specs=in_specs,
        out_specs=out_specs,
        out_shape=out_shape,
    )


@functools.lru_cache(maxsize=None)
def _tc_conv_tanh(fin, gw, hout):
    def body(z, g, wc, rt, b, inv, o):
        o[...] = jnp.tanh(_conv_base(z, g, wc, rt, b, inv))
    return _tc_call(body, 1, fin, gw, hout)


@functools.lru_cache(maxsize=None)
def _tc_conv_plain(fin, gw, hout):
    def body(z, g, wc, rt, b, inv, o):
        o[...] = _conv_base(z, g, wc, rt, b, inv)
    return _tc_call(body, 1, fin, gw, hout)


@functools.lru_cache(maxsize=None)
def _tc_conv_rk4(fin, gw, hout, ci, wi):
    def body(z, g, wc, rt, b, inv, h, accp, oz, oa):
        k = _conv_base(z, g, wc, rt, b, inv)
        oz[...] = h[...] + ci * k
        oa[...] = accp[...] + wi * k
    return _tc_call(body, 2, fin, gw, hout, extra_row_ins=2)


@functools.lru_cache(maxsize=None)
def _tc_conv2_logsoftmax(fin, gw, hout, ncls):
    def body(z, g, wc, rt, b, inv, o):
        t = jnp.tanh(_conv_base(z, g, wc, rt, b, inv))
        col = lax.broadcasted_iota(jnp.int32, t.shape, 1)
        m = col < ncls
        tm = jnp.where(m, t, jnp.float32(-1e30))
        mx = jnp.max(tm, axis=1, keepdims=True)
        ex = jnp.where(m, jnp.exp(t - mx), 0.0)
        lse = jnp.log(jnp.sum(ex, axis=1, keepdims=True))
        o[...] = t - mx - lse
    return _tc_call(body, 1, fin, gw, hout)


def _tc_invdeg():
    def body(a_ref, b_ref, o_ref):
        cnt = (b_ref[...] - a_ref[...]).astype(jnp.float32)
        o_ref[...] = 1.0 / jnp.maximum(cnt, 1.0)
    return pl.pallas_call(
        body,
        grid=(NP // BR,),
        in_specs=[pl.BlockSpec((BR, 1), lambda i: (i, 0))] * 2,
        out_specs=pl.BlockSpec((BR, 1), lambda i: (i, 0)),
        out_shape=jax.ShapeDtypeStruct((NP, 1), jnp.float32),
    )


def kernel(x, edge_index, edge_attr, W1, root1, b1, Wa, roota, ba,
           Wb, rootb, bb, W2, root2, b2):
    E = edge_index.shape[1]
    src = edge_index[0].astype(jnp.int32)
    dst = edge_index[1].astype(jnp.int32)
    u = edge_attr[:, 0]

    dst_s, src_s, u_s = lax.sort((dst, src, u), num_keys=1)

    KPAD = 1024
    Ep = ((E + KPAD - 1) // KPAD) * KPAD
    pad = Ep - E
    src_p = jnp.pad(src_s, (0, pad))
    u_p = jnp.pad(u_s, (0, pad))

    cnt = jnp.zeros((N_NODES,), jnp.int32).at[dst].add(1)
    seg = jnp.concatenate([jnp.zeros((1,), jnp.int32),
                           jnp.cumsum(cnt, dtype=jnp.int32)])
    seg_p = jnp.pad(seg, (0, NP + 8 - (N_NODES + 1)),
                    constant_values=E)

    seg_a = jnp.pad(seg[:N_NODES], (0, NP - N_NODES)).reshape(NP, 1)
    seg_b = jnp.pad(seg[1:], (0, NP - N_NODES)).reshape(NP, 1)
    invdeg = _tc_invdeg()(seg_a, seg_b)

    xp = jnp.pad(x, ((0, NP - N_NODES), (0, 0)))

    D, H = x.shape[1], W1.shape[2]
    NCLS = W2.shape[2]
    HP = 128
    wc1 = jnp.concatenate([W1[0], W1[1]], axis=0)
    wca = jnp.concatenate([Wa[0], Wa[1]], axis=0)
    wcb = jnp.concatenate([Wb[0], Wb[1]], axis=0)
    wc2 = jnp.pad(jnp.concatenate([W2[0], W2[1]], axis=0),
                  ((0, 0), (0, HP - NCLS)))
    root2p = jnp.pad(root2, ((0, 0), (0, HP - NCLS)))
    b2p = jnp.pad(b2, (0, HP - NCLS)).reshape(1, HP)
    b1r = b1.reshape(1, H)
    bar = ba.reshape(1, H)
    bbr = bb.reshape(1, H)

    sc_d = _seg_sums_sc(D, 256)
    sc_h = _seg_sums_sc(H, 1024)
    edges = (src_p, u_p, seg_p)

    g = sc_d(xp, *edges)
    h = _tc_conv_tanh(D, 2 * D, H)(xp, g, wc1, root1, b1r, invdeg)

    def conv_a(z):
        gz = sc_h(z, *edges)
        return _tc_conv_plain(H, 2 * H, H)(z, gz, wca, roota, bar, invdeg)

    def conv_b_rk4(z, accp, ci, wi):
        gz = sc_h(z, *edges)
        return _tc_conv_rk4(H, 2 * H, H, ci, wi)(
            z, gz, wcb, rootb, bbr, invdeg, h, accp)

    z2, acc1 = conv_b_rk4(conv_a(h), h, 0.5 * HSTEP, HSTEP / 6.0)
    z3, acc2 = conv_b_rk4(conv_a(z2), acc1, 0.5 * HSTEP, 2.0 * HSTEP / 6.0)
    z4, acc3 = conv_b_rk4(conv_a(z3), acc2, HSTEP, 2.0 * HSTEP / 6.0)
    _, hfin = conv_b_rk4(conv_a(z4), acc3, 0.0, HSTEP / 6.0)

    g2 = sc_h(hfin, *edges)
    out = _tc_conv2_logsoftmax(H, 2 * H, HP, NCLS)(
        hfin, g2, wc2, root2p, b2p, invdeg)
    return out[:N_NODES, :NCLS]


# speedup vs baseline: 15.4438x; 1.0911x over previous
import dataclasses
import functools

import jax
import jax.numpy as jnp
from jax import lax
from jax.experimental import pallas as pl
from jax.experimental.pallas import tpu as pltpu
from jax.experimental.pallas import tpu_sc as plsc

N_NODES = 10000
NP = 10240
R_TILE = 320
N_WORKERS = 32
HSTEP = 3.0
BR = 1024


@functools.lru_cache(maxsize=None)
def _seg_sums_sc(F, K, prefetch=True):
    assert K & (K - 1) == 0
    LOGK = K.bit_length() - 1
    NG = F // 16
    NBUF = 2 if prefetch else 1
    mesh = plsc.VectorSubcoreMesh(core_axis_name="c", subcore_axis_name="s")
    cp = pltpu.CompilerParams(needs_layout_passes=False,
                              use_tc_tiling_on_sc=False)

    @functools.partial(
        pl.kernel,
        out_type=jax.ShapeDtypeStruct((NP, 2 * F), jnp.float32),
        mesh=mesh,
        compiler_params=cp,
        scratch_types=[
            pltpu.VMEM((NBUF * K,), jnp.int32),
            pltpu.VMEM((NBUF * K, F), jnp.float32),
            pltpu.VMEM((R_TILE, 2 * F), jnp.float32),
            pltpu.VMEM((NBUF * K + 16,), jnp.float32),
            pltpu.VMEM((R_TILE + 24,), jnp.int32),
            pltpu.SemaphoreType.DMA,
        ],
    )
    def sc_kernel(z_hbm, src_hbm, u_hbm, seg_hbm, g_hbm,
                  srcv, zbuf, acc, useg, segs, sem):
        wid = lax.axis_index("s") * 2 + lax.axis_index("c")
        r0 = wid * R_TILE
        pltpu.sync_copy(seg_hbm.at[pl.ds(r0, R_TILE + 8)],
                        segs.at[pl.ds(0, R_TILE + 8)])

        zeros16 = jnp.zeros((16,), jnp.float32)

        def slot_of(cj):
            return (cj & 1) * K if prefetch else cj * 0

        def issue(cj):
            base = cj * K
            slot = slot_of(cj)
            pltpu.sync_copy(src_hbm.at[pl.ds(base, K)],
                            srcv.at[pl.ds(slot, K)])
            pltpu.sync_copy(u_hbm.at[pl.ds(base, K)],
                            useg.at[pl.ds(slot, K)])
            pltpu.async_copy(z_hbm.at[srcv.at[pl.ds(slot, K)]],
                             zbuf.at[pl.ds(slot, K)], sem)

        s0 = segs[pl.ds(0, 16)][0]
        if prefetch:
            issue(lax.shift_right_logical(s0, LOGK))

        def edge_body(off):
            def body(j, accs):
                el = j + off
                uvec = plsc.load_gather(useg, [jnp.broadcast_to(el, (16,))])
                new = []
                for g in range(NG):
                    zv = zbuf[el, pl.ds(g * 16, 16)]
                    new.append(accs[g] + zv)
                for g in range(NG):
                    zv = zbuf[el, pl.ds(g * 16, 16)]
                    new.append(accs[NG + g] + uvec * zv)
                return tuple(new)
            return body

        def row_body(r, carry):
            loaded, s_cur = carry
            e2 = segs[pl.ds(r + 1, 16)][0]

            def chunk_step(state):
                j, loaded_in = state[0], state[1]
                accs = state[2:]
                cj = lax.shift_right_logical(j, LOGK)
                base = cj * K

                @pl.when(cj != loaded_in)
                def _():
                    if prefetch:
                        pltpu.make_async_copy(
                            z_hbm.at[srcv.at[pl.ds(slot_of(cj), K)]],
                            zbuf.at[pl.ds(slot_of(cj), K)], sem).wait()
                        issue(cj + 1)
                    else:
                        issue(cj)
                        pltpu.make_async_copy(
                            z_hbm.at[srcv.at[pl.ds(slot_of(cj), K)]],
                            zbuf.at[pl.ds(slot_of(cj), K)], sem).wait()

                off = slot_of(cj) - base
                jend = jnp.minimum(e2, base + K)
                accs = lax.fori_loop(j, jend, edge_body(off), accs)
                return (jend, cj) + accs

            init = (s_cur, loaded) + (zeros16,) * (2 * NG)
            out = lax.while_loop(lambda st: st[0] < e2, chunk_step, init)
            for g in range(2 * NG):
                acc[r, pl.ds(g * 16, 16)] = out[2 + g]
            return out[1], e2

        fin = lax.fori_loop(0, R_TILE, row_body, (jnp.int32(-1), s0))
        if prefetch:
            c0 = lax.shift_right_logical(s0, LOGK)
            pend = jnp.where(fin[0] < 0, c0, fin[0] + 1)
            pltpu.make_async_copy(
                z_hbm.at[srcv.at[pl.ds(slot_of(pend), K)]],
                zbuf.at[pl.ds(slot_of(pend), K)], sem).wait()
        pltpu.sync_copy(acc, g_hbm.at[pl.ds(r0, R_TILE)])

    return sc_kernel


def _mm(a, b):
    ah = a.astype(jnp.bfloat16)
    al = (a - ah.astype(jnp.float32)).astype(jnp.bfloat16)
    bh = b.astype(jnp.bfloat16)
    bl = (b - bh.astype(jnp.float32)).astype(jnp.bfloat16)
    d = functools.partial(jnp.dot, preferred_element_type=jnp.float32)
    return d(ah, bh) + (d(al, bh) + d(ah, bl))


def _conv_base(z_ref, g_ref, wc_ref, rt_ref, b_ref, inv_ref):
    return (inv_ref[...] * _mm(g_ref[...], wc_ref[...])
            + _mm(z_ref[...], rt_ref[...]) + b_ref[...])


def _row_spec(width):
    return pl.BlockSpec((BR, width), lambda i: (i, 0))


def _full_spec(shape):
    return pl.BlockSpec(shape, lambda i: (0, 0))


def _tc_call(body, n_out, fin, gw, hout, extra_row_ins=0):
    in_specs = [
        _row_spec(fin),
        _row_spec(gw),
        _full_spec((gw, hout)),
        _full_spec((fin, hout)),
        _full_spec((1, hout)),
        pl.BlockSpec((BR, 1), lambda i: (i, 0)),
    ] + [_row_spec(hout)] * extra_row_ins
    out_shape = [jax.ShapeDtypeStruct((NP, hout), jnp.float32)] * n_out
    out_specs = [_row_spec(hout)] * n_out
    if n_out == 1:
        out_shape, out_specs = out_shape[0], out_specs[0]
    return pl.pallas_call(
        body,
        grid=(NP // BR,),
        in_specs=in_specs,
        out_specs=out_specs,
        out_shape=out_shape,
    )


@functools.lru_cache(maxsize=None)
def _tc_conv_tanh(fin, gw, hout):
    def body(z, g, wc, rt, b, inv, o):
        o[...] = jnp.tanh(_conv_base(z, g, wc, rt, b, inv))
    return _tc_call(body, 1, fin, gw, hout)


@functools.lru_cache(maxsize=None)
def _tc_conv_plain(fin, gw, hout):
    def body(z, g, wc, rt, b, inv, o):
        o[...] = _conv_base(z, g, wc, rt, b, inv)
    return _tc_call(body, 1, fin, gw, hout)


@functools.lru_cache(maxsize=None)
def _tc_conv_rk4(fin, gw, hout, ci, wi):
    def body(z, g, wc, rt, b, inv, h, accp, oz, oa):
        k = _conv_base(z, g, wc, rt, b, inv)
        oz[...] = h[...] + ci * k
        oa[...] = accp[...] + wi * k
    return _tc_call(body, 2, fin, gw, hout, extra_row_ins=2)


@functools.lru_cache(maxsize=None)
def _tc_conv2_logsoftmax(fin, gw, hout, ncls):
    def body(z, g, wc, rt, b, inv, o):
        t = jnp.tanh(_conv_base(z, g, wc, rt, b, inv))
        col = lax.broadcasted_iota(jnp.int32, t.shape, 1)
        m = col < ncls
        tm = jnp.where(m, t, jnp.float32(-1e30))
        mx = jnp.max(tm, axis=1, keepdims=True)
        ex = jnp.where(m, jnp.exp(t - mx), 0.0)
        lse = jnp.log(jnp.sum(ex, axis=1, keepdims=True))
        o[...] = t - mx - lse
    return _tc_call(body, 1, fin, gw, hout)


def _tc_invdeg():
    def body(a_ref, b_ref, o_ref):
        cnt = (b_ref[...] - a_ref[...]).astype(jnp.float32)
        o_ref[...] = 1.0 / jnp.maximum(cnt, 1.0)
    return pl.pallas_call(
        body,
        grid=(NP // BR,),
        in_specs=[pl.BlockSpec((BR, 1), lambda i: (i, 0))] * 2,
        out_specs=pl.BlockSpec((BR, 1), lambda i: (i, 0)),
        out_shape=jax.ShapeDtypeStruct((NP, 1), jnp.float32),
    )


def kernel(x, edge_index, edge_attr, W1, root1, b1, Wa, roota, ba,
           Wb, rootb, bb, W2, root2, b2):
    E = edge_index.shape[1]
    src = edge_index[0].astype(jnp.int32)
    dst = edge_index[1].astype(jnp.int32)
    u = edge_attr[:, 0]

    dst_s, src_s, u_s = lax.sort((dst, src, u), num_keys=1)

    KPAD = 1024
    Ep = ((E + KPAD - 1) // KPAD + 1) * KPAD
    pad = Ep - E
    src_p = jnp.pad(src_s, (0, pad))
    u_p = jnp.pad(u_s, (0, pad))

    cnt = jnp.zeros((N_NODES,), jnp.int32).at[dst].add(1)
    seg = jnp.concatenate([jnp.zeros((1,), jnp.int32),
                           jnp.cumsum(cnt, dtype=jnp.int32)])
    seg_p = jnp.pad(seg, (0, NP + 8 - (N_NODES + 1)),
                    constant_values=E)

    seg_a = jnp.pad(seg[:N_NODES], (0, NP - N_NODES)).reshape(NP, 1)
    seg_b = jnp.pad(seg[1:], (0, NP - N_NODES)).reshape(NP, 1)
    invdeg = _tc_invdeg()(seg_a, seg_b)

    xp = jnp.pad(x, ((0, NP - N_NODES), (0, 0)))

    D, H = x.shape[1], W1.shape[2]
    NCLS = W2.shape[2]
    HP = 128
    wc1 = jnp.concatenate([W1[0], W1[1] - W1[0]], axis=0)
    wca = jnp.concatenate([Wa[0], Wa[1] - Wa[0]], axis=0)
    wcb = jnp.concatenate([Wb[0], Wb[1] - Wb[0]], axis=0)
    wc2 = jnp.pad(jnp.concatenate([W2[0], W2[1] - W2[0]], axis=0),
                  ((0, 0), (0, HP - NCLS)))
    root2p = jnp.pad(root2, ((0, 0), (0, HP - NCLS)))
    b2p = jnp.pad(b2, (0, HP - NCLS)).reshape(1, HP)
    b1r = b1.reshape(1, H)
    bar = ba.reshape(1, H)
    bbr = bb.reshape(1, H)

    sc_d = _seg_sums_sc(D, 256, prefetch=False)
    sc_h = _seg_sums_sc(H, 512, prefetch=True)
    edges = (src_p, u_p, seg_p)

    g = sc_d(xp, *edges)
    h = _tc_conv_tanh(D, 2 * D, H)(xp, g, wc1, root1, b1r, invdeg)

    def conv_a(z):
        gz = sc_h(z, *edges)
        return _tc_conv_plain(H, 2 * H, H)(z, gz, wca, roota, bar, invdeg)

    def conv_b_rk4(z, accp, ci, wi):
        gz = sc_h(z, *edges)
        return _tc_conv_rk4(H, 2 * H, H, ci, wi)(
            z, gz, wcb, rootb, bbr, invdeg, h, accp)

    z2, acc1 = conv_b_rk4(conv_a(h), h, 0.5 * HSTEP, HSTEP / 6.0)
    z3, acc2 = conv_b_rk4(conv_a(z2), acc1, 0.5 * HSTEP, 2.0 * HSTEP / 6.0)
    z4, acc3 = conv_b_rk4(conv_a(z3), acc2, HSTEP, 2.0 * HSTEP / 6.0)
    _, hfin = conv_b_rk4(conv_a(z4), acc3, 0.0, HSTEP / 6.0)

    g2 = sc_h(hfin, *edges)
    out = _tc_conv2_logsoftmax(H, 2 * H, HP, NCLS)(
        hfin, g2, wc2, root2p, b2p, invdeg)
    return out[:N_NODES, :NCLS]
